# baseline (speedup 1.0000x reference)
import functools

import jax
import jax.numpy as jnp
from jax import lax
from jax.experimental import pallas as pl
from jax.experimental.pallas import tpu as pltpu
from jax.experimental.pallas import tpu_sc as plsc

_S, _D, _U, _E, _K = 256, 256, 512, 64, 2
_G = 8
_NG = _E // _G
_NCHUNK = _E // 16


def _logits_kernel(x_ref, wr_ref, out_ref):
    lgt = jax.lax.dot_general(
        wr_ref[...], x_ref[...], (((1,), (1,)), ((), ())),
        preferred_element_type=jnp.float32,
    )
    for w in range(_S // 16):
        out_ref[w] = lgt[:, w * 16:(w + 1) * 16]


def _sc_routing_kernel(logits_hbm, wsel_hbm, logits_v, wsel_v):
    wid = lax.axis_index("s") * 2 + lax.axis_index("c")

    @pl.when(wid < 16)
    def _work():
        base = wid * 16
        pltpu.sync_copy(logits_hbm.at[wid], logits_v)
        nbig = jnp.full((16,), -1e30, jnp.float32)
        zero_i = jnp.zeros((16,), jnp.int32)
        m1 = nbig
        m2 = nbig
        i1 = zero_i
        i2 = zero_i
        for e in range(_E):
            v = logits_v[e, :]
            ev = jnp.full((16,), e, jnp.int32)
            gt1 = v > m1
            gt2 = v > m2
            m2 = jnp.where(gt1, m1, jnp.where(gt2, v, m2))
            i2 = jnp.where(gt1, i1, jnp.where(gt2, ev, i2))
            m1 = jnp.where(gt1, v, m1)
            i1 = jnp.where(gt1, ev, i1)
        onev = jnp.full((16,), 1.0, jnp.float32)
        w1 = onev / (onev + jnp.exp(m2 - m1))
        w2 = onev - w1
        zero_f = jnp.zeros((16,), jnp.float32)
        for e in range(_E):
            ev = jnp.full((16,), e, jnp.int32)
            wsel_v[e, :] = jnp.where(
                i1 == ev, w1, jnp.where(i2 == ev, w2, zero_f)
            )
        pltpu.sync_copy(wsel_v, wsel_hbm.at[wid])


def _moe_kernel(x_ref, wsel_in_ref, wu_ref, wd_ref, bu_ref, bd_ref, out_ref,
                wsel_ref):
    g = pl.program_id(0)
    x = x_ref[...]

    @pl.when(g == 0)
    def _regroup():
        wsel = wsel_in_ref[...]
        for ng in range(_NG):
            wsel_ref[ng] = wsel[:, ng * _G:(ng + 1) * _G]

    wstep = wsel_ref[g]
    acc = None
    for j in range(_G):
        h = jax.lax.dot_general(
            x, wu_ref[j], (((1,), (1,)), ((), ())),
            preferred_element_type=jnp.float32,
        )
        h = h + bu_ref[j]
        h = 0.5 * h * (1.0 + jax.lax.erf(h * 0.7071067811865476))
        y = jax.lax.dot_general(
            h, wd_ref[j], (((1,), (1,)), ((), ())),
            preferred_element_type=jnp.float32,
        )
        y = y + bd_ref[j]
        contrib = y * wstep[:, j:j + 1]
        acc = contrib if acc is None else acc + contrib

    @pl.when(g == 0)
    def _init():
        out_ref[...] = acc

    @pl.when(g != 0)
    def _acc():
        out_ref[...] += acc


def kernel(x, W_router, W_up, W_down, b_up, b_down):
    b, s, d = x.shape
    x2 = x.reshape(s, d)

    logits_t = pl.pallas_call(
        _logits_kernel,
        out_shape=jax.ShapeDtypeStruct((_S // 16, _E, 16), jnp.float32),
    )(x2, W_router)

    sc_route = functools.partial(
        pl.kernel,
        mesh=plsc.VectorSubcoreMesh(core_axis_name="c", subcore_axis_name="s"),
        out_type=jax.ShapeDtypeStruct((_S // 16, _E, 16), jnp.float32),
        scratch_types=[
            pltpu.VMEM((_E, 16), jnp.float32),
            pltpu.VMEM((_E, 16), jnp.float32),
        ],
    )(_sc_routing_kernel)
    wsel3 = sc_route(logits_t)
    wsel = wsel3.transpose(0, 2, 1).reshape(_S, _E)

    bu3 = b_up.reshape(_E, 1, _U)
    bd3 = b_down.reshape(_E, 1, _D)

    out = pl.pallas_call(
        _moe_kernel,
        grid=(_NG,),
        in_specs=[
            pl.BlockSpec((_S, _D), lambda g: (0, 0)),
            pl.BlockSpec((_S, _E), lambda g: (0, 0)),
            pl.BlockSpec((_G, _U, _D), lambda g: (g, 0, 0)),
            pl.BlockSpec((_G, _D, _U), lambda g: (g, 0, 0)),
            pl.BlockSpec((_G, 1, _U), lambda g: (g, 0, 0)),
            pl.BlockSpec((_G, 1, _D), lambda g: (g, 0, 0)),
        ],
        out_specs=pl.BlockSpec((_S, _D), lambda g: (0, 0)),
        out_shape=jax.ShapeDtypeStruct((_S, _D), jnp.float32),
        scratch_shapes=[pltpu.VMEM((_NG, _S, _G), jnp.float32)],
        compiler_params=pltpu.CompilerParams(
            dimension_semantics=("arbitrary",),
        ),
    )(x2, wsel, W_up, W_down, bu3, bd3)

    return out.reshape(b, s, d)


# speedup vs baseline: 1.5171x; 1.5171x over previous
import jax
import jax.numpy as jnp
from jax.experimental import pallas as pl
from jax.experimental.pallas import tpu as pltpu

_S, _D, _U, _E, _K = 256, 256, 512, 64, 2
_G = 8
_NG = _E // _G


def _moe_kernel(x_ref, wr_ref, wu_ref, wd_ref, bu_ref, bd_ref, out_ref,
                wsel_ref):
    c = pl.program_id(0)
    gi = pl.program_id(1)
    g = c * (_NG // 2) + gi
    x = x_ref[...]

    @pl.when(gi == 0)
    def _route():
        wr = wr_ref[...]
        logits = jax.lax.dot_general(
            x, wr, (((1,), (1,)), ((), ())),
            preferred_element_type=jnp.float32,
        )
        e_iota = jax.lax.broadcasted_iota(jnp.int32, logits.shape, 1)
        i1 = jnp.argmax(logits, axis=1)
        m1 = jnp.max(logits, axis=1, keepdims=True)
        masked = jnp.where(e_iota == i1[:, None], -jnp.inf, logits)
        i2 = jnp.argmax(masked, axis=1)
        m2 = jnp.max(masked, axis=1, keepdims=True)
        w1 = jax.nn.sigmoid(m1 - m2)
        w2 = 1.0 - w1
        wsel = jnp.where(e_iota == i1[:, None], w1, 0.0) + jnp.where(
            e_iota == i2[:, None], w2, 0.0
        )
        for ng in range(_NG):
            wsel_ref[ng] = wsel[:, ng * _G:(ng + 1) * _G]

    wstep = wsel_ref[g]
    acc = None
    for j in range(_G):
        h = jax.lax.dot_general(
            x, wu_ref[j], (((1,), (1,)), ((), ())),
            preferred_element_type=jnp.float32,
        )
        h = h + bu_ref[j]
        h = 0.5 * h * (1.0 + jax.lax.erf(h * 0.7071067811865476))
        y = jax.lax.dot_general(
            h, wd_ref[j], (((1,), (1,)), ((), ())),
            preferred_element_type=jnp.float32,
        )
        y = y + bd_ref[j]
        contrib = y * wstep[:, j:j + 1]
        acc = contrib if acc is None else acc + contrib

    @pl.when(gi == 0)
    def _init():
        out_ref[0] = acc

    @pl.when(gi != 0)
    def _acc():
        out_ref[0] += acc


def _combine_kernel(p_ref, out_ref):
    out_ref[...] = p_ref[0] + p_ref[1]


def kernel(x, W_router, W_up, W_down, b_up, b_down):
    b, s, d = x.shape
    x2 = x.reshape(s, d)

    bu3 = b_up.reshape(_E, 1, _U)
    bd3 = b_down.reshape(_E, 1, _D)

    nh = _NG // 2
    partial = pl.pallas_call(
        _moe_kernel,
        grid=(2, nh),
        in_specs=[
            pl.BlockSpec((_S, _D), lambda c, gi: (0, 0)),
            pl.BlockSpec((_E, _D), lambda c, gi: (0, 0)),
            pl.BlockSpec((_G, _U, _D), lambda c, gi: (c * nh + gi, 0, 0)),
            pl.BlockSpec((_G, _D, _U), lambda c, gi: (c * nh + gi, 0, 0)),
            pl.BlockSpec((_G, 1, _U), lambda c, gi: (c * nh + gi, 0, 0)),
            pl.BlockSpec((_G, 1, _D), lambda c, gi: (c * nh + gi, 0, 0)),
        ],
        out_specs=pl.BlockSpec((1, _S, _D), lambda c, gi: (c, 0, 0)),
        out_shape=jax.ShapeDtypeStruct((2, _S, _D), jnp.float32),
        scratch_shapes=[pltpu.VMEM((_NG, _S, _G), jnp.float32)],
        compiler_params=pltpu.CompilerParams(
            dimension_semantics=("parallel", "arbitrary"),
        ),
    )(x2, W_router, W_up, W_down, bu3, bd3)

    out = pl.pallas_call(
        _combine_kernel,
        out_shape=jax.ShapeDtypeStruct((_S, _D), jnp.float32),
    )(partial)

    return out.reshape(b, s, d)


# speedup vs baseline: 1.6519x; 1.0888x over previous
import jax
import jax.numpy as jnp
from jax.experimental import pallas as pl
from jax.experimental.pallas import tpu as pltpu

_S, _D, _U, _E, _K = 256, 256, 512, 64, 2
_G = 8
_NG = _E // _G


def _moe_kernel(x_ref, wr_ref, wu_ref, wd_ref, bu_ref, bd_ref, out_ref,
                wsel_ref):
    g = pl.program_id(0)
    x = x_ref[...]

    @pl.when(g == 0)
    def _route():
        wr = wr_ref[...]
        logits = jax.lax.dot_general(
            x, wr, (((1,), (1,)), ((), ())),
            preferred_element_type=jnp.float32,
        )
        e_iota = jax.lax.broadcasted_iota(jnp.int32, logits.shape, 1)
        i1 = jnp.argmax(logits, axis=1)
        m1 = jnp.max(logits, axis=1, keepdims=True)
        masked = jnp.where(e_iota == i1[:, None], -jnp.inf, logits)
        i2 = jnp.argmax(masked, axis=1)
        m2 = jnp.max(masked, axis=1, keepdims=True)
        w1 = jax.nn.sigmoid(m1 - m2)
        w2 = 1.0 - w1
        wsel = jnp.where(e_iota == i1[:, None], w1, 0.0) + jnp.where(
            e_iota == i2[:, None], w2, 0.0
        )
        for ng in range(_NG):
            wsel_ref[ng] = wsel[:, ng * _G:(ng + 1) * _G]

    wstep = wsel_ref[g]
    acc = None
    for j in range(_G):
        h = jax.lax.dot_general(
            x, wu_ref[j], (((1,), (1,)), ((), ())),
            preferred_element_type=jnp.float32,
        )
        h = h + bu_ref[j]
        h = 0.5 * h * (1.0 + jax.lax.erf(h * 0.7071067811865476))
        y = jax.lax.dot_general(
            h, wd_ref[j], (((1,), (1,)), ((), ())),
            preferred_element_type=jnp.float32,
        )
        y = y + bd_ref[j]
        contrib = y * wstep[:, j:j + 1]
        acc = contrib if acc is None else acc + contrib

    @pl.when(g == 0)
    def _init():
        out_ref[...] = acc

    @pl.when(g != 0)
    def _acc():
        out_ref[...] += acc


def kernel(x, W_router, W_up, W_down, b_up, b_down):
    b, s, d = x.shape
    x2 = x.reshape(s, d)

    bu3 = b_up.reshape(_E, 1, _U)
    bd3 = b_down.reshape(_E, 1, _D)

    out = pl.pallas_call(
        _moe_kernel,
        grid=(_NG,),
        in_specs=[
            pl.BlockSpec((_S, _D), lambda g: (0, 0)),
            pl.BlockSpec((_E, _D), lambda g: (0, 0)),
            pl.BlockSpec((_G, _U, _D), lambda g: (g, 0, 0)),
            pl.BlockSpec((_G, _D, _U), lambda g: (g, 0, 0)),
            pl.BlockSpec((_G, 1, _U), lambda g: (g, 0, 0)),
            pl.BlockSpec((_G, 1, _D), lambda g: (g, 0, 0)),
        ],
        out_specs=pl.BlockSpec((_S, _D), lambda g: (0, 0)),
        out_shape=jax.ShapeDtypeStruct((_S, _D), jnp.float32),
        scratch_shapes=[pltpu.VMEM((_NG, _S, _G), jnp.float32)],
        compiler_params=pltpu.CompilerParams(
            dimension_semantics=("arbitrary",),
        ),
    )(x2, W_router, W_up, W_down, bu3, bd3)

    return out.reshape(b, s, d)
